# Initial kernel scaffold; baseline (speedup 1.0000x reference)
import jax
import jax.numpy as jnp
from jax.experimental import pallas as pl


def kernel(x, W_enc, W_dec, pre_bias, latent_bias):
    raise NotImplementedError("write your pallas kernel here")


# speedup vs baseline: 3.8036x; 3.8036x over previous
import functools

import jax
import jax.numpy as jnp
from jax import lax
from jax.experimental import pallas as pl
from jax.experimental.pallas import tpu as pltpu

K = 64


def _encode_body(x_ref, w_ref, pb_ref, lb_ref, out_ref):
    xb = x_ref[...] - pb_ref[...]
    acts = lax.dot_general(xb, w_ref[...], (((1,), (1,)), ((), ())),
                           preferred_element_type=jnp.float32)
    out_ref[...] = acts + lb_ref[...]


def _encode(x, W_enc, pre_bias, latent_bias, *, br=1024, bh=2048):
    n, d = x.shape
    h = W_enc.shape[0]
    grid = (n // br, h // bh)
    return pl.pallas_call(
        _encode_body,
        grid=grid,
        in_specs=[
            pl.BlockSpec((br, d), lambda r, hh: (r, 0)),
            pl.BlockSpec((bh, d), lambda r, hh: (hh, 0)),
            pl.BlockSpec((1, d), lambda r, hh: (0, 0)),
            pl.BlockSpec((1, bh), lambda r, hh: (0, hh)),
        ],
        out_specs=pl.BlockSpec((br, bh), lambda r, hh: (r, hh)),
        out_shape=jax.ShapeDtypeStruct((n, h), jnp.float32),
    )(x, W_enc, pre_bias.reshape(1, d), latent_bias.reshape(1, h))


def _topk_body(acts_ref, sc_ref, tv_ref, ti_ref, a_ref):
    r, h = acts_ref.shape
    relu = jnp.maximum(acts_ref[...], 0.0)
    a_ref[...] = relu
    iota = lax.broadcasted_iota(jnp.int32, (r, h), 1)

    ms, idxs = [], []
    for _ in range(K):
        av = a_ref[...]
        m = jnp.max(av, axis=1, keepdims=True)
        eq = av == m
        idx = jnp.min(jnp.where(eq, iota, h), axis=1, keepdims=True)
        ms.append(m)
        idxs.append(idx)
        a_ref[...] = jnp.where(iota == idx, -1.0, av)
    tv_ref[...] = jnp.concatenate(ms, axis=1)
    ti_ref[...] = jnp.concatenate(idxs, axis=1)
    sc_ref[...] = jnp.where(a_ref[...] < -0.5, relu, 0.0)


def _topk(pre_acts, *, br=128):
    n, h = pre_acts.shape
    grid = (n // br,)
    return pl.pallas_call(
        _topk_body,
        grid=grid,
        in_specs=[pl.BlockSpec((br, h), lambda r: (r, 0))],
        out_specs=[
            pl.BlockSpec((br, h), lambda r: (r, 0)),
            pl.BlockSpec((br, K), lambda r: (r, 0)),
            pl.BlockSpec((br, K), lambda r: (r, 0)),
        ],
        out_shape=[
            jax.ShapeDtypeStruct((n, h), jnp.float32),
            jax.ShapeDtypeStruct((n, K), jnp.float32),
            jax.ShapeDtypeStruct((n, K), jnp.int32),
        ],
        scratch_shapes=[pltpu.VMEM((br, h), jnp.float32)],
    )(pre_acts)


def _decode_body(sc_ref, wd_ref, pb_ref, out_ref):
    kk = pl.program_id(1)
    nk = pl.num_programs(1)
    part = lax.dot_general(sc_ref[...], wd_ref[...], (((1,), (1,)), ((), ())),
                           preferred_element_type=jnp.float32)

    @pl.when(kk == 0)
    def _():
        out_ref[...] = part + pb_ref[...]

    @pl.when(kk != 0)
    def _():
        out_ref[...] = out_ref[...] + part


def _decode(sparse_code, W_dec, pre_bias, *, br=1024, bk=2048):
    n, h = sparse_code.shape
    d = W_dec.shape[0]
    grid = (n // br, h // bk)
    return pl.pallas_call(
        _decode_body,
        grid=grid,
        in_specs=[
            pl.BlockSpec((br, bk), lambda r, kk: (r, kk)),
            pl.BlockSpec((d, bk), lambda r, kk: (0, kk)),
            pl.BlockSpec((1, d), lambda r, kk: (0, 0)),
        ],
        out_specs=pl.BlockSpec((br, d), lambda r, kk: (r, 0)),
        out_shape=jax.ShapeDtypeStruct((n, d), jnp.float32),
        compiler_params=pltpu.CompilerParams(
            dimension_semantics=("parallel", "arbitrary")),
    )(sparse_code, W_dec, pre_bias.reshape(1, d))


def kernel(x, W_enc, W_dec, pre_bias, latent_bias):
    pre_acts = _encode(x, W_enc, pre_bias, latent_bias)
    sparse_code, topk_values, topk_indices = _topk(pre_acts)
    reconstruction = _decode(sparse_code, W_dec, pre_bias)
    return (reconstruction, sparse_code, pre_acts, topk_values, topk_indices)
